# Initial kernel scaffold; baseline (speedup 1.0000x reference)
import jax
import jax.numpy as jnp
from jax.experimental import pallas as pl


def kernel(f, edge_index_cp, edge_index_hc, hc_ids, salary, W_fc, attn_W, attn_b, out_W, out_b):
    raise NotImplementedError("write your pallas kernel here")


# speedup vs baseline: 17.7499x; 17.7499x over previous
import functools
import jax
import jax.numpy as jnp
from jax import lax
from jax.experimental import pallas as pl
from jax.experimental.pallas import tpu as pltpu
from jax.experimental.pallas import tpu_sc as plsc

N = 10000
D = 128
E = 320000
NC = 2
NS = 16
NW = NC * NS
EW = E // NW
C = 80
NCH = EW // C
R = 1000


def _mm_body(x_ref, w_ref, a_ref, b_ref, z_ref, s_ref):
    z = jnp.maximum(jnp.dot(x_ref[...], w_ref[...],
                            preferred_element_type=jnp.float32), 0.0)
    z_ref[...] = z
    s = jnp.dot(z, a_ref[...], preferred_element_type=jnp.float32)
    col = lax.broadcasted_iota(jnp.int32, s.shape, 1)
    bv = b_ref[...]
    s_ref[...] = s + jnp.where(col == 0, bv[0, 0], 0.0)


def _mm1(f, W, A2, b2):
    return pl.pallas_call(
        _mm_body,
        grid=(N // R,),
        in_specs=[
            pl.BlockSpec((R, D), lambda i: (i, 0)),
            pl.BlockSpec((D, D), lambda i: (0, 0)),
            pl.BlockSpec((D, 2), lambda i: (0, 0)),
            pl.BlockSpec((1, 2), lambda i: (0, 0)),
        ],
        out_specs=[
            pl.BlockSpec((R, D), lambda i: (i, 0)),
            pl.BlockSpec((R, 2), lambda i: (i, 0)),
        ],
        out_shape=[
            jax.ShapeDtypeStruct((N, D), jnp.float32),
            jax.ShapeDtypeStruct((N, 2), jnp.float32),
        ],
    )(f, W, A2, b2)


def _fin_body(aggp_ref, denp_ref, z_ref, w_ref, b_ref, zo_ref, so_ref):
    den = denp_ref[0] + denp_ref[1]
    agg = aggp_ref[0] + aggp_ref[1]
    pos = den > 0.0
    dsafe = jnp.where(pos, den, 1.0)
    zn = jnp.where(pos, jnp.maximum(agg / dsafe, 0.0), z_ref[...])
    zo_ref[...] = zn
    s = jnp.dot(zn, w_ref[...], preferred_element_type=jnp.float32)
    col = lax.broadcasted_iota(jnp.int32, s.shape, 1)
    bv = b_ref[...]
    so_ref[...] = s + jnp.where(col == 0, bv[0, 0], 0.0)


def _fin(aggp, denp, zprev, W2, b2):
    return pl.pallas_call(
        _fin_body,
        grid=(N // R,),
        in_specs=[
            pl.BlockSpec((2, R, D), lambda i: (0, i, 0)),
            pl.BlockSpec((2, R, 1), lambda i: (0, i, 0)),
            pl.BlockSpec((R, D), lambda i: (i, 0)),
            pl.BlockSpec((D, 2), lambda i: (0, 0)),
            pl.BlockSpec((1, 2), lambda i: (0, 0)),
        ],
        out_specs=[
            pl.BlockSpec((R, D), lambda i: (i, 0)),
            pl.BlockSpec((R, 2), lambda i: (i, 0)),
        ],
        out_shape=[
            jax.ShapeDtypeStruct((N, D), jnp.float32),
            jax.ShapeDtypeStruct((N, 2), jnp.float32),
        ],
    )(aggp, denp, zprev, W2, b2)


_SC_MESH = plsc.VectorSubcoreMesh(core_axis_name="c", subcore_axis_name="s")

_RPT = 624
_RLAST = _RPT * (NS - 1)
_DPT = 1000


def _chunks(total):
    off = 0
    while off < total:
        sz = min(C, total - off)
        yield off, sz
        off += sz


def _edge_body(ei_hbm, s12_hbm, z_hbm,
               agg_out, den_out,
               s12_v, src_v, dst_v, ex_v, rows_v, agg_sh, den_sh, sem):
    c = lax.axis_index("c")
    s = lax.axis_index("s")

    zv = jnp.zeros((16,), jnp.float32)

    def zrow(i, carry):
        for j in range(D // 16):
            rows_v[i, pl.ds(j * 16, 16)] = zv
        return carry

    lax.fori_loop(0, C, zrow, 0)

    def zex(g, carry):
        ex_v[pl.ds(g * 16, 16)] = zv
        return carry

    lax.fori_loop(0, C // 16, zex, 0)

    @pl.when(s < NS - 1)
    def _():
        for o, sz in _chunks(_RPT):
            pltpu.sync_copy(rows_v.at[pl.ds(0, sz)],
                            agg_sh.at[pl.ds(s * _RPT + o, sz)])

    @pl.when(s == NS - 1)
    def _():
        for o, sz in _chunks(N - _RLAST):
            pltpu.sync_copy(rows_v.at[pl.ds(0, sz)],
                            agg_sh.at[pl.ds(_RLAST + o, sz)])

    @pl.when(s < N // _DPT)
    def _():
        for o, sz in _chunks(_DPT):
            pltpu.sync_copy(ex_v.at[pl.ds(0, sz)],
                            den_sh.at[pl.ds(s * _DPT + o, sz)])

    pltpu.sync_copy(s12_hbm, s12_v)
    plsc.subcore_barrier()

    base = c * (E // NC) + s * EW

    def chunk(k, carry):
        off = base + k * C
        pltpu.sync_copy(ei_hbm.at[pl.ds(off, C)], src_v)
        pltpu.sync_copy(ei_hbm.at[pl.ds(E + off, C)], dst_v)
        cp = pltpu.async_copy(z_hbm.at[dst_v], rows_v, sem)

        def grp(g, carry2):
            sl = pl.ds(g * 16, 16)
            srcv = src_v[sl]
            dstv = dst_v[sl]
            s1 = plsc.load_gather(s12_v, [srcv * 2])
            s2 = plsc.load_gather(s12_v, [dstv * 2 + 1])
            e = s1 + s2
            e = jnp.maximum(e, e * 0.01)
            ex_v[sl] = jnp.exp(e)
            return carry2

        lax.fori_loop(0, C // 16, grp, 0)
        pltpu.sync_copy(ex_v, den_sh.at[src_v], add=True)
        cp.wait()

        def rowg(g, carry2):
            exv = ex_v[pl.ds(g * 16, 16)]
            r0 = g * 16
            for l in range(16):
                exi = exv[l]
                for j in range(D // 16):
                    slj = pl.ds(j * 16, 16)
                    rows_v[r0 + l, slj] = rows_v[r0 + l, slj] * exi
            return carry2

        lax.fori_loop(0, C // 16, rowg, 0)
        pltpu.sync_copy(rows_v, agg_sh.at[src_v], add=True)
        return carry

    lax.fori_loop(0, NCH, chunk, 0)
    plsc.subcore_barrier()

    @pl.when(s < NS - 1)
    def _():
        for o, sz in _chunks(_RPT):
            pltpu.sync_copy(agg_sh.at[pl.ds(s * _RPT + o, sz)],
                            rows_v.at[pl.ds(0, sz)])
            pltpu.sync_copy(rows_v.at[pl.ds(0, sz)],
                            agg_out.at[c, pl.ds(s * _RPT + o, sz)])

    @pl.when(s == NS - 1)
    def _():
        for o, sz in _chunks(N - _RLAST):
            pltpu.sync_copy(agg_sh.at[pl.ds(_RLAST + o, sz)],
                            rows_v.at[pl.ds(0, sz)])
            pltpu.sync_copy(rows_v.at[pl.ds(0, sz)],
                            agg_out.at[c, pl.ds(_RLAST + o, sz)])

    @pl.when(s < N // _DPT)
    def _():
        for o, sz in _chunks(_DPT):
            pltpu.sync_copy(den_sh.at[pl.ds(s * _DPT + o, sz)],
                            ex_v.at[pl.ds(0, sz)])
            pltpu.sync_copy(ex_v.at[pl.ds(0, sz)],
                            den_out.at[pl.ds(c * N + s * _DPT + o, sz)])


@functools.partial(
    pl.kernel,
    out_type=(
        jax.ShapeDtypeStruct((NC, N, D), jnp.float32),
        jax.ShapeDtypeStruct((NC * N,), jnp.float32),
    ),
    mesh=_SC_MESH,
    compiler_params=pltpu.CompilerParams(needs_layout_passes=False),
    scratch_types=[
        pltpu.VMEM((2 * N,), jnp.float32),
        pltpu.VMEM((C,), jnp.int32),
        pltpu.VMEM((C,), jnp.int32),
        pltpu.VMEM((C,), jnp.float32),
        pltpu.VMEM((C, D), jnp.float32),
        pltpu.VMEM_SHARED((N, D), jnp.float32),
        pltpu.VMEM_SHARED((N,), jnp.float32),
        pltpu.SemaphoreType.DMA,
    ],
)
def _edge_pass(*refs):
    _edge_body(*refs)


_HPW = 1024 // NW


def _readout_body(t_hbm, hc_hbm, sal_hbm, prm_hbm, y_hbm,
                  t_v, hc_v, sal_v, prm_v, y_v):
    c = lax.axis_index("c")
    s = lax.axis_index("s")
    w = s * NC + c
    pltpu.sync_copy(t_hbm, t_v)
    pltpu.sync_copy(hc_hbm.at[pl.ds(w * _HPW, _HPW)], hc_v)
    pltpu.sync_copy(sal_hbm.at[pl.ds(w * _HPW, _HPW)], sal_v)
    pltpu.sync_copy(prm_hbm, prm_v)
    pv = prm_v[...]
    ws0 = pv[0]
    ws1 = pv[1]
    b0 = pv[2]
    b1 = pv[3]
    for g in range(_HPW // 16):
        sl = pl.ds(g * 16, 16)
        hc = hc_v[sl]
        sal = sal_v[sl]
        t0 = plsc.load_gather(t_v, [hc * 2])
        t1 = plsc.load_gather(t_v, [hc * 2 + 1])
        r0 = jnp.maximum(t0 + sal * ws0 + b0, 0.0)
        r1 = jnp.maximum(t1 + sal * ws1 + b1, 0.0)
        e0 = jnp.exp(r0)
        e1 = jnp.exp(r1)
        inv = 1.0 / (e0 + e1)
        idx = lax.iota(jnp.int32, 16) * 2 + g * 32
        plsc.store_scatter(y_v, [idx], e0 * inv)
        plsc.store_scatter(y_v, [idx + 1], e1 * inv)
    pltpu.sync_copy(y_v, y_hbm.at[pl.ds(w * 2 * _HPW, 2 * _HPW)])


@functools.partial(
    pl.kernel,
    out_type=jax.ShapeDtypeStruct((2048,), jnp.float32),
    mesh=_SC_MESH,
    compiler_params=pltpu.CompilerParams(needs_layout_passes=False),
    scratch_types=[
        pltpu.VMEM((2 * N,), jnp.float32),
        pltpu.VMEM((_HPW,), jnp.int32),
        pltpu.VMEM((_HPW,), jnp.float32),
        pltpu.VMEM((16,), jnp.float32),
        pltpu.VMEM((2 * _HPW,), jnp.float32),
    ],
)
def _readout(*refs):
    _readout_body(*refs)


def kernel(f, edge_index_cp, edge_index_hc, hc_ids, salary, W_fc,
           attn_W, attn_b, out_W, out_b):
    A2 = attn_W.reshape(2, D).T
    b2 = jnp.concatenate([attn_b, jnp.zeros((1,), jnp.float32)]).reshape(1, 2)
    zb2 = jnp.zeros((1, 2), jnp.float32)

    z0, s12 = _mm1(f, W_fc, A2, b2)
    aggp, denp = _edge_pass(edge_index_cp.reshape(-1), s12.reshape(-1), z0)
    z1, s12b = _fin(aggp, denp.reshape(NC, N, 1), z0, A2, b2)
    aggp2, denp2 = _edge_pass(edge_index_hc.reshape(-1), s12b.reshape(-1), z1)
    z2, t = _fin(aggp2, denp2.reshape(NC, N, 1), z1, out_W[:D], zb2)

    prm = jnp.concatenate([out_W[D], out_b, jnp.zeros((12,), jnp.float32)])
    yflat = _readout(t.reshape(-1), hc_ids, salary.reshape(-1), prm)
    return z2, yflat.reshape(-1, 2)


# speedup vs baseline: 30.6362x; 1.7260x over previous
import functools
import jax
import jax.numpy as jnp
from jax import lax
from jax.experimental import pallas as pl
from jax.experimental.pallas import tpu as pltpu
from jax.experimental.pallas import tpu_sc as plsc

N = 10000
D = 128
E = 320000
NC = 2
NS = 16
NW = NC * NS
EW = E // NW
C = 80
NCH = EW // C
R = 1000


def _mm_body(x_ref, w_ref, a_ref, b_ref, z_ref, s_ref):
    z = jnp.maximum(jnp.dot(x_ref[...], w_ref[...],
                            preferred_element_type=jnp.float32), 0.0)
    z_ref[...] = z
    s = jnp.dot(z, a_ref[...], preferred_element_type=jnp.float32)
    col = lax.broadcasted_iota(jnp.int32, s.shape, 1)
    bv = b_ref[...]
    s_ref[...] = s + jnp.where(col == 0, bv[0, 0], 0.0)


def _mm1(f, W, A2, b2):
    return pl.pallas_call(
        _mm_body,
        grid=(N // R,),
        in_specs=[
            pl.BlockSpec((R, D), lambda i: (i, 0)),
            pl.BlockSpec((D, D), lambda i: (0, 0)),
            pl.BlockSpec((D, 2), lambda i: (0, 0)),
            pl.BlockSpec((1, 2), lambda i: (0, 0)),
        ],
        out_specs=[
            pl.BlockSpec((R, D), lambda i: (i, 0)),
            pl.BlockSpec((R, 2), lambda i: (i, 0)),
        ],
        out_shape=[
            jax.ShapeDtypeStruct((N, D), jnp.float32),
            jax.ShapeDtypeStruct((N, 2), jnp.float32),
        ],
    )(f, W, A2, b2)


def _fin_body(aggp_ref, denp_ref, z_ref, w_ref, b_ref, zo_ref, so_ref):
    den = denp_ref[0] + denp_ref[1]
    agg = aggp_ref[0] + aggp_ref[1]
    pos = den > 0.0
    dsafe = jnp.where(pos, den, 1.0)
    zn = jnp.where(pos, jnp.maximum(agg / dsafe, 0.0), z_ref[...])
    zo_ref[...] = zn
    s = jnp.dot(zn, w_ref[...], preferred_element_type=jnp.float32)
    col = lax.broadcasted_iota(jnp.int32, s.shape, 1)
    bv = b_ref[...]
    so_ref[...] = s + jnp.where(col == 0, bv[0, 0], 0.0)


def _fin(aggp, denp, zprev, W2, b2):
    return pl.pallas_call(
        _fin_body,
        grid=(N // R,),
        in_specs=[
            pl.BlockSpec((2, R, D), lambda i: (0, i, 0)),
            pl.BlockSpec((2, R, 1), lambda i: (0, i, 0)),
            pl.BlockSpec((R, D), lambda i: (i, 0)),
            pl.BlockSpec((D, 2), lambda i: (0, 0)),
            pl.BlockSpec((1, 2), lambda i: (0, 0)),
        ],
        out_specs=[
            pl.BlockSpec((R, D), lambda i: (i, 0)),
            pl.BlockSpec((R, 2), lambda i: (i, 0)),
        ],
        out_shape=[
            jax.ShapeDtypeStruct((N, D), jnp.float32),
            jax.ShapeDtypeStruct((N, 2), jnp.float32),
        ],
    )(aggp, denp, zprev, W2, b2)


_SC_MESH = plsc.VectorSubcoreMesh(core_axis_name="c", subcore_axis_name="s")

_RPT = 624
_RLAST = _RPT * (NS - 1)
_DPT = 1000


def _chunks(total):
    off = 0
    while off < total:
        sz = min(C, total - off)
        yield off, sz
        off += sz


def _edge_body(ei_hbm, s12_hbm, z_hbm,
               agg_out, den_out,
               s12_v, srcA, dstA, srcB, dstB, ex_v, rowsA, rowsB,
               agg_sh, den_sh, semA, semB, semI):
    c = lax.axis_index("c")
    s = lax.axis_index("s")
    rows_v = rowsA

    zv = jnp.zeros((16,), jnp.float32)

    def zrow(i, carry):
        for j in range(D // 16):
            rows_v[i, pl.ds(j * 16, 16)] = zv
        return carry

    lax.fori_loop(0, C, zrow, 0)

    def zex(g, carry):
        ex_v[pl.ds(g * 16, 16)] = zv
        return carry

    lax.fori_loop(0, C // 16, zex, 0)

    @pl.when(s < NS - 1)
    def _():
        for o, sz in _chunks(_RPT):
            pltpu.sync_copy(rows_v.at[pl.ds(0, sz)],
                            agg_sh.at[pl.ds(s * _RPT + o, sz)])

    @pl.when(s == NS - 1)
    def _():
        for o, sz in _chunks(N - _RLAST):
            pltpu.sync_copy(rows_v.at[pl.ds(0, sz)],
                            agg_sh.at[pl.ds(_RLAST + o, sz)])

    @pl.when(s < N // _DPT)
    def _():
        for o, sz in _chunks(_DPT):
            pltpu.sync_copy(ex_v.at[pl.ds(0, sz)],
                            den_sh.at[pl.ds(s * _DPT + o, sz)])

    pltpu.sync_copy(s12_hbm, s12_v)
    plsc.subcore_barrier()

    base = c * (E // NC) + s * EW
    sets = ((srcA, dstA, rowsA, semA), (srcB, dstB, rowsB, semB))

    def process(k, b, start_next):
        srcb, dstb, rowsb, semb = sets[b]
        if start_next:
            srcn, dstn, rowsn, semn = sets[1 - b]
            offn = base + (k + 1) * C
            ia = pltpu.async_copy(ei_hbm.at[pl.ds(offn, C)], srcn, semI)
            ib = pltpu.async_copy(ei_hbm.at[pl.ds(E + offn, C)], dstn, semI)
        pltpu.make_async_copy(z_hbm.at[pl.ds(0, C)], rowsb, semb).wait()

        def grp(g, carry2):
            sl = pl.ds(g * 16, 16)
            srcv = srcb[sl]
            dstv = dstb[sl]
            s1 = plsc.load_gather(s12_v, [srcv * 2])
            s2 = plsc.load_gather(s12_v, [dstv * 2 + 1])
            e = s1 + s2
            e = jnp.maximum(e, e * 0.01)
            ex_v[sl] = jnp.exp(e)
            return carry2

        lax.fori_loop(0, C // 16, grp, 0)
        pltpu.sync_copy(ex_v, den_sh.at[srcb], add=True)
        if start_next:
            ia.wait()
            ib.wait()
            pltpu.async_copy(z_hbm.at[dstn], rowsn, semn)

        def rowg(g, carry2):
            exv = ex_v[pl.ds(g * 16, 16)]
            r0 = g * 16
            for l in range(16):
                exi = exv[l]
                for j in range(D // 16):
                    slj = pl.ds(j * 16, 16)
                    rowsb[r0 + l, slj] = rowsb[r0 + l, slj] * exi
            return carry2

        lax.fori_loop(0, C // 16, rowg, 0)
        pltpu.sync_copy(rowsb, agg_sh.at[srcb], add=True)

    pltpu.sync_copy(ei_hbm.at[pl.ds(base, C)], srcA)
    pltpu.sync_copy(ei_hbm.at[pl.ds(E + base, C)], dstA)
    pltpu.async_copy(z_hbm.at[dstA], rowsA, semA)

    def pair(g, carry):
        process(2 * g, 0, True)
        process(2 * g + 1, 1, True)
        return carry

    lax.fori_loop(0, NCH // 2, pair, 0)
    process(NCH - 1, 0, False)
    plsc.subcore_barrier()

    @pl.when(s < NS - 1)
    def _():
        for o, sz in _chunks(_RPT):
            pltpu.sync_copy(agg_sh.at[pl.ds(s * _RPT + o, sz)],
                            rows_v.at[pl.ds(0, sz)])
            pltpu.sync_copy(rows_v.at[pl.ds(0, sz)],
                            agg_out.at[c, pl.ds(s * _RPT + o, sz)])

    @pl.when(s == NS - 1)
    def _():
        for o, sz in _chunks(N - _RLAST):
            pltpu.sync_copy(agg_sh.at[pl.ds(_RLAST + o, sz)],
                            rows_v.at[pl.ds(0, sz)])
            pltpu.sync_copy(rows_v.at[pl.ds(0, sz)],
                            agg_out.at[c, pl.ds(_RLAST + o, sz)])

    @pl.when(s < N // _DPT)
    def _():
        for o, sz in _chunks(_DPT):
            pltpu.sync_copy(den_sh.at[pl.ds(s * _DPT + o, sz)],
                            ex_v.at[pl.ds(0, sz)])
            pltpu.sync_copy(ex_v.at[pl.ds(0, sz)],
                            den_out.at[pl.ds(c * N + s * _DPT + o, sz)])


@functools.partial(
    pl.kernel,
    out_type=(
        jax.ShapeDtypeStruct((NC, N, D), jnp.float32),
        jax.ShapeDtypeStruct((NC * N,), jnp.float32),
    ),
    mesh=_SC_MESH,
    compiler_params=pltpu.CompilerParams(needs_layout_passes=False),
    scratch_types=[
        pltpu.VMEM((2 * N,), jnp.float32),
        pltpu.VMEM((C,), jnp.int32),
        pltpu.VMEM((C,), jnp.int32),
        pltpu.VMEM((C,), jnp.int32),
        pltpu.VMEM((C,), jnp.int32),
        pltpu.VMEM((C,), jnp.float32),
        pltpu.VMEM((C, D), jnp.float32),
        pltpu.VMEM((C, D), jnp.float32),
        pltpu.VMEM_SHARED((N, D), jnp.float32),
        pltpu.VMEM_SHARED((N,), jnp.float32),
        pltpu.SemaphoreType.DMA,
        pltpu.SemaphoreType.DMA,
        pltpu.SemaphoreType.DMA,
    ],
)
def _edge_pass(*refs):
    _edge_body(*refs)


_HPW = 1024 // NW


def _readout_body(t_hbm, hc_hbm, sal_hbm, prm_hbm, y_hbm,
                  t_v, hc_v, sal_v, prm_v, y_v):
    c = lax.axis_index("c")
    s = lax.axis_index("s")
    w = s * NC + c
    pltpu.sync_copy(t_hbm, t_v)
    pltpu.sync_copy(hc_hbm.at[pl.ds(w * _HPW, _HPW)], hc_v)
    pltpu.sync_copy(sal_hbm.at[pl.ds(w * _HPW, _HPW)], sal_v)
    pltpu.sync_copy(prm_hbm, prm_v)
    pv = prm_v[...]
    ws0 = pv[0]
    ws1 = pv[1]
    b0 = pv[2]
    b1 = pv[3]
    for g in range(_HPW // 16):
        sl = pl.ds(g * 16, 16)
        hc = hc_v[sl]
        sal = sal_v[sl]
        t0 = plsc.load_gather(t_v, [hc * 2])
        t1 = plsc.load_gather(t_v, [hc * 2 + 1])
        r0 = jnp.maximum(t0 + sal * ws0 + b0, 0.0)
        r1 = jnp.maximum(t1 + sal * ws1 + b1, 0.0)
        e0 = jnp.exp(r0)
        e1 = jnp.exp(r1)
        inv = 1.0 / (e0 + e1)
        idx = lax.iota(jnp.int32, 16) * 2 + g * 32
        plsc.store_scatter(y_v, [idx], e0 * inv)
        plsc.store_scatter(y_v, [idx + 1], e1 * inv)
    pltpu.sync_copy(y_v, y_hbm.at[pl.ds(w * 2 * _HPW, 2 * _HPW)])


@functools.partial(
    pl.kernel,
    out_type=jax.ShapeDtypeStruct((2048,), jnp.float32),
    mesh=_SC_MESH,
    compiler_params=pltpu.CompilerParams(needs_layout_passes=False),
    scratch_types=[
        pltpu.VMEM((2 * N,), jnp.float32),
        pltpu.VMEM((_HPW,), jnp.int32),
        pltpu.VMEM((_HPW,), jnp.float32),
        pltpu.VMEM((16,), jnp.float32),
        pltpu.VMEM((2 * _HPW,), jnp.float32),
    ],
)
def _readout(*refs):
    _readout_body(*refs)


def kernel(f, edge_index_cp, edge_index_hc, hc_ids, salary, W_fc,
           attn_W, attn_b, out_W, out_b):
    A2 = attn_W.reshape(2, D).T
    b2 = jnp.concatenate([attn_b, jnp.zeros((1,), jnp.float32)]).reshape(1, 2)
    zb2 = jnp.zeros((1, 2), jnp.float32)

    z0, s12 = _mm1(f, W_fc, A2, b2)
    aggp, denp = _edge_pass(edge_index_cp.reshape(-1), s12.reshape(-1), z0)
    z1, s12b = _fin(aggp, denp.reshape(NC, N, 1), z0, A2, b2)
    aggp2, denp2 = _edge_pass(edge_index_hc.reshape(-1), s12b.reshape(-1), z1)
    z2, t = _fin(aggp2, denp2.reshape(NC, N, 1), z1, out_W[:D], zb2)

    prm = jnp.concatenate([out_W[D], out_b, jnp.zeros((12,), jnp.float32)])
    yflat = _readout(t.reshape(-1), hc_ids, salary.reshape(-1), prm)
    return z2, yflat.reshape(-1, 2)


# speedup vs baseline: 32.5694x; 1.0631x over previous
import functools
import jax
import jax.numpy as jnp
from jax import lax
from jax.experimental import pallas as pl
from jax.experimental.pallas import tpu as pltpu
from jax.experimental.pallas import tpu_sc as plsc

N = 10000
D = 128
E = 320000
NC = 2
NS = 16
NW = NC * NS
EW = E // NW
C = 80
NCH = EW // C
R = 1000


def _mm_body(x_ref, w_ref, a_ref, b_ref, z_ref, s_ref):
    z = jnp.maximum(jnp.dot(x_ref[...], w_ref[...],
                            preferred_element_type=jnp.float32), 0.0)
    z_ref[...] = z
    s = jnp.dot(z, a_ref[...], preferred_element_type=jnp.float32)
    col = lax.broadcasted_iota(jnp.int32, s.shape, 1)
    bv = b_ref[...]
    s_ref[...] = s + jnp.where(col == 0, bv[0, 0], 0.0)


def _mm1(f, W, A2, b2):
    return pl.pallas_call(
        _mm_body,
        grid=(N // R,),
        in_specs=[
            pl.BlockSpec((R, D), lambda i: (i, 0)),
            pl.BlockSpec((D, D), lambda i: (0, 0)),
            pl.BlockSpec((D, 2), lambda i: (0, 0)),
            pl.BlockSpec((1, 2), lambda i: (0, 0)),
        ],
        out_specs=[
            pl.BlockSpec((R, D), lambda i: (i, 0)),
            pl.BlockSpec((R, 2), lambda i: (i, 0)),
        ],
        out_shape=[
            jax.ShapeDtypeStruct((N, D), jnp.float32),
            jax.ShapeDtypeStruct((N, 2), jnp.float32),
        ],
    )(f, W, A2, b2)


def _fin_body(aggp_ref, denp_ref, z_ref, w_ref, b_ref, zo_ref, so_ref):
    den = denp_ref[0] + denp_ref[1]
    agg = aggp_ref[0] + aggp_ref[1]
    pos = den > 0.0
    dsafe = jnp.where(pos, den, 1.0)
    zn = jnp.where(pos, jnp.maximum(agg / dsafe, 0.0), z_ref[...])
    zo_ref[...] = zn
    s = jnp.dot(zn, w_ref[...], preferred_element_type=jnp.float32)
    col = lax.broadcasted_iota(jnp.int32, s.shape, 1)
    bv = b_ref[...]
    so_ref[...] = s + jnp.where(col == 0, bv[0, 0], 0.0)


def _fin(aggp, denp, zprev, W2, b2):
    return pl.pallas_call(
        _fin_body,
        grid=(N // R,),
        in_specs=[
            pl.BlockSpec((2, R, D), lambda i: (0, i, 0)),
            pl.BlockSpec((2, R, 1), lambda i: (0, i, 0)),
            pl.BlockSpec((R, D), lambda i: (i, 0)),
            pl.BlockSpec((D, 2), lambda i: (0, 0)),
            pl.BlockSpec((1, 2), lambda i: (0, 0)),
        ],
        out_specs=[
            pl.BlockSpec((R, D), lambda i: (i, 0)),
            pl.BlockSpec((R, 2), lambda i: (i, 0)),
        ],
        out_shape=[
            jax.ShapeDtypeStruct((N, D), jnp.float32),
            jax.ShapeDtypeStruct((N, 2), jnp.float32),
        ],
    )(aggp, denp, zprev, W2, b2)


_SC_MESH = plsc.VectorSubcoreMesh(core_axis_name="c", subcore_axis_name="s")

_RPT = 624
_RLAST = _RPT * (NS - 1)
_DPT = 1000


def _chunks(total):
    off = 0
    while off < total:
        sz = min(C, total - off)
        yield off, sz
        off += sz


def _edge_body(ei_hbm, s12_hbm, z_hbm,
               agg_out, den_out,
               s12_v, srcA, dstA, srcB, dstB, srcSA, srcSB, exA, exB,
               rowsA, rowsB, agg_sh, den_sh,
               semA, semB, semI, semD0, semD1, semS0, semS1):
    c = lax.axis_index("c")
    s = lax.axis_index("s")
    rows_v = rowsA
    ex_v = exA

    zv = jnp.zeros((16,), jnp.float32)

    def zrow(i, carry):
        for j in range(D // 16):
            rows_v[i, pl.ds(j * 16, 16)] = zv
        return carry

    lax.fori_loop(0, C, zrow, 0)

    def zex(g, carry):
        ex_v[pl.ds(g * 16, 16)] = zv
        return carry

    lax.fori_loop(0, C // 16, zex, 0)

    @pl.when(s < NS - 1)
    def _():
        for o, sz in _chunks(_RPT):
            pltpu.sync_copy(rows_v.at[pl.ds(0, sz)],
                            agg_sh.at[pl.ds(s * _RPT + o, sz)])

    @pl.when(s == NS - 1)
    def _():
        for o, sz in _chunks(N - _RLAST):
            pltpu.sync_copy(rows_v.at[pl.ds(0, sz)],
                            agg_sh.at[pl.ds(_RLAST + o, sz)])

    @pl.when(s < N // _DPT)
    def _():
        for o, sz in _chunks(_DPT):
            pltpu.sync_copy(ex_v.at[pl.ds(0, sz)],
                            den_sh.at[pl.ds(s * _DPT + o, sz)])

    pltpu.sync_copy(s12_hbm, s12_v)
    plsc.subcore_barrier()

    base = c * (E // NC) + s * EW
    sets = (
        (srcA, dstA, srcSA, exA, rowsA, semA, semD0, semS0),
        (srcB, dstB, srcSB, exB, rowsB, semB, semD1, semS1),
    )

    def process(k, b, start_next):
        srcb, dstb, srcSb, exb, rowsb, semb, semDb, semSb = sets[b]
        if start_next:
            srcn, dstn, srcSn, exn, rowsn, semn, semDn, semSn = sets[1 - b]
            offn = base + (k + 1) * C
            ia = pltpu.async_copy(ei_hbm.at[pl.ds(offn, C)], srcn, semI)
            ib = pltpu.async_copy(ei_hbm.at[pl.ds(E + offn, C)], dstn, semI)
        pltpu.make_async_copy(z_hbm.at[pl.ds(0, C)], rowsb, semb).wait()

        @pl.when(k >= 2)
        def _():
            pltpu.make_async_copy(s12_hbm.at[pl.ds(0, C)], exb, semDb).wait()

        def grp(g, carry2):
            sl = pl.ds(g * 16, 16)
            srcv = srcb[sl]
            dstv = dstb[sl]
            srcSb[sl] = srcv
            s1 = plsc.load_gather(s12_v, [srcv * 2])
            s2 = plsc.load_gather(s12_v, [dstv * 2 + 1])
            e = s1 + s2
            e = jnp.maximum(e, e * 0.01)
            exb[sl] = jnp.exp(e)
            return carry2

        lax.fori_loop(0, C // 16, grp, 0)
        pltpu.async_copy(exb, den_sh.at[srcSb], semDb, add=True)
        if start_next:
            ia.wait()
            ib.wait()

            @pl.when(k >= 1)
            def _():
                pltpu.make_async_copy(z_hbm.at[pl.ds(0, C)], rowsn,
                                      semSn).wait()

            pltpu.async_copy(z_hbm.at[dstn], rowsn, semn)

        def rowg(g, carry2):
            exv = exb[pl.ds(g * 16, 16)]
            r0 = g * 16
            for l in range(16):
                exi = exv[l]
                for j in range(D // 16):
                    slj = pl.ds(j * 16, 16)
                    rowsb[r0 + l, slj] = rowsb[r0 + l, slj] * exi
            return carry2

        lax.fori_loop(0, C // 16, rowg, 0)
        pltpu.async_copy(rowsb, agg_sh.at[srcSb], semSb, add=True)

    pltpu.sync_copy(ei_hbm.at[pl.ds(base, C)], srcA)
    pltpu.sync_copy(ei_hbm.at[pl.ds(E + base, C)], dstA)
    pltpu.async_copy(z_hbm.at[dstA], rowsA, semA)

    def pair(g, carry):
        process(2 * g, 0, True)
        process(2 * g + 1, 1, True)
        return carry

    lax.fori_loop(0, NCH // 2, pair, 0)
    process(NCH - 1, 0, False)

    pltpu.make_async_copy(s12_hbm.at[pl.ds(0, C)], exA, semD0).wait()
    pltpu.make_async_copy(s12_hbm.at[pl.ds(0, C)], exB, semD1).wait()
    pltpu.make_async_copy(z_hbm.at[pl.ds(0, C)], rowsA, semS0).wait()
    pltpu.make_async_copy(z_hbm.at[pl.ds(0, C)], rowsB, semS1).wait()
    plsc.subcore_barrier()

    @pl.when(s < NS - 1)
    def _():
        for o, sz in _chunks(_RPT):
            pltpu.sync_copy(agg_sh.at[pl.ds(s * _RPT + o, sz)],
                            rows_v.at[pl.ds(0, sz)])
            pltpu.sync_copy(rows_v.at[pl.ds(0, sz)],
                            agg_out.at[c, pl.ds(s * _RPT + o, sz)])

    @pl.when(s == NS - 1)
    def _():
        for o, sz in _chunks(N - _RLAST):
            pltpu.sync_copy(agg_sh.at[pl.ds(_RLAST + o, sz)],
                            rows_v.at[pl.ds(0, sz)])
            pltpu.sync_copy(rows_v.at[pl.ds(0, sz)],
                            agg_out.at[c, pl.ds(_RLAST + o, sz)])

    @pl.when(s < N // _DPT)
    def _():
        for o, sz in _chunks(_DPT):
            pltpu.sync_copy(den_sh.at[pl.ds(s * _DPT + o, sz)],
                            ex_v.at[pl.ds(0, sz)])
            pltpu.sync_copy(ex_v.at[pl.ds(0, sz)],
                            den_out.at[pl.ds(c * N + s * _DPT + o, sz)])


@functools.partial(
    pl.kernel,
    out_type=(
        jax.ShapeDtypeStruct((NC, N, D), jnp.float32),
        jax.ShapeDtypeStruct((NC * N,), jnp.float32),
    ),
    mesh=_SC_MESH,
    compiler_params=pltpu.CompilerParams(needs_layout_passes=False),
    scratch_types=[
        pltpu.VMEM((2 * N,), jnp.float32),
        pltpu.VMEM((C,), jnp.int32),
        pltpu.VMEM((C,), jnp.int32),
        pltpu.VMEM((C,), jnp.int32),
        pltpu.VMEM((C,), jnp.int32),
        pltpu.VMEM((C,), jnp.int32),
        pltpu.VMEM((C,), jnp.int32),
        pltpu.VMEM((C,), jnp.float32),
        pltpu.VMEM((C,), jnp.float32),
        pltpu.VMEM((C, D), jnp.float32),
        pltpu.VMEM((C, D), jnp.float32),
        pltpu.VMEM_SHARED((N, D), jnp.float32),
        pltpu.VMEM_SHARED((N,), jnp.float32),
        pltpu.SemaphoreType.DMA,
        pltpu.SemaphoreType.DMA,
        pltpu.SemaphoreType.DMA,
        pltpu.SemaphoreType.DMA,
        pltpu.SemaphoreType.DMA,
        pltpu.SemaphoreType.DMA,
        pltpu.SemaphoreType.DMA,
    ],
)
def _edge_pass(*refs):
    _edge_body(*refs)


_HPW = 1024 // NW


def _readout_body(t_hbm, hc_hbm, sal_hbm, prm_hbm, y_hbm,
                  t_v, hc_v, sal_v, prm_v, y_v):
    c = lax.axis_index("c")
    s = lax.axis_index("s")
    w = s * NC + c
    pltpu.sync_copy(t_hbm, t_v)
    pltpu.sync_copy(hc_hbm.at[pl.ds(w * _HPW, _HPW)], hc_v)
    pltpu.sync_copy(sal_hbm.at[pl.ds(w * _HPW, _HPW)], sal_v)
    pltpu.sync_copy(prm_hbm, prm_v)
    pv = prm_v[...]
    ws0 = pv[0]
    ws1 = pv[1]
    b0 = pv[2]
    b1 = pv[3]
    for g in range(_HPW // 16):
        sl = pl.ds(g * 16, 16)
        hc = hc_v[sl]
        sal = sal_v[sl]
        t0 = plsc.load_gather(t_v, [hc * 2])
        t1 = plsc.load_gather(t_v, [hc * 2 + 1])
        r0 = jnp.maximum(t0 + sal * ws0 + b0, 0.0)
        r1 = jnp.maximum(t1 + sal * ws1 + b1, 0.0)
        e0 = jnp.exp(r0)
        e1 = jnp.exp(r1)
        inv = 1.0 / (e0 + e1)
        idx = lax.iota(jnp.int32, 16) * 2 + g * 32
        plsc.store_scatter(y_v, [idx], e0 * inv)
        plsc.store_scatter(y_v, [idx + 1], e1 * inv)
    pltpu.sync_copy(y_v, y_hbm.at[pl.ds(w * 2 * _HPW, 2 * _HPW)])


@functools.partial(
    pl.kernel,
    out_type=jax.ShapeDtypeStruct((2048,), jnp.float32),
    mesh=_SC_MESH,
    compiler_params=pltpu.CompilerParams(needs_layout_passes=False),
    scratch_types=[
        pltpu.VMEM((2 * N,), jnp.float32),
        pltpu.VMEM((_HPW,), jnp.int32),
        pltpu.VMEM((_HPW,), jnp.float32),
        pltpu.VMEM((16,), jnp.float32),
        pltpu.VMEM((2 * _HPW,), jnp.float32),
    ],
)
def _readout(*refs):
    _readout_body(*refs)


def kernel(f, edge_index_cp, edge_index_hc, hc_ids, salary, W_fc,
           attn_W, attn_b, out_W, out_b):
    A2 = attn_W.reshape(2, D).T
    b2 = jnp.concatenate([attn_b, jnp.zeros((1,), jnp.float32)]).reshape(1, 2)
    zb2 = jnp.zeros((1, 2), jnp.float32)

    z0, s12 = _mm1(f, W_fc, A2, b2)
    aggp, denp = _edge_pass(edge_index_cp.reshape(-1), s12.reshape(-1), z0)
    z1, s12b = _fin(aggp, denp.reshape(NC, N, 1), z0, A2, b2)
    aggp2, denp2 = _edge_pass(edge_index_hc.reshape(-1), s12b.reshape(-1), z1)
    z2, t = _fin(aggp2, denp2.reshape(NC, N, 1), z1, out_W[:D], zb2)

    prm = jnp.concatenate([out_W[D], out_b, jnp.zeros((12,), jnp.float32)])
    yflat = _readout(t.reshape(-1), hc_ids, salary.reshape(-1), prm)
    return z2, yflat.reshape(-1, 2)


# speedup vs baseline: 35.0274x; 1.0755x over previous
import functools
import jax
import jax.numpy as jnp
from jax import lax
from jax.experimental import pallas as pl
from jax.experimental.pallas import tpu as pltpu
from jax.experimental.pallas import tpu_sc as plsc

N = 10000
D = 128
E = 320000
NC = 2
NS = 16
NW = NC * NS
EW = E // NW
C = 64
NCH = EW // C
TAIL = EW - NCH * C
R = 1000


def _mm_body(x_ref, w_ref, a_ref, b_ref, z_ref, s_ref):
    z = jnp.maximum(jnp.dot(x_ref[...], w_ref[...],
                            preferred_element_type=jnp.float32), 0.0)
    z_ref[...] = z
    s = jnp.dot(z, a_ref[...], preferred_element_type=jnp.float32)
    col = lax.broadcasted_iota(jnp.int32, s.shape, 1)
    bv = b_ref[...]
    s_ref[...] = s + jnp.where(col == 0, bv[0, 0], 0.0)


def _mm1(f, W, A2, b2):
    return pl.pallas_call(
        _mm_body,
        grid=(N // R,),
        in_specs=[
            pl.BlockSpec((R, D), lambda i: (i, 0)),
            pl.BlockSpec((D, D), lambda i: (0, 0)),
            pl.BlockSpec((D, 2), lambda i: (0, 0)),
            pl.BlockSpec((1, 2), lambda i: (0, 0)),
        ],
        out_specs=[
            pl.BlockSpec((R, D), lambda i: (i, 0)),
            pl.BlockSpec((R, 2), lambda i: (i, 0)),
        ],
        out_shape=[
            jax.ShapeDtypeStruct((N, D), jnp.float32),
            jax.ShapeDtypeStruct((N, 2), jnp.float32),
        ],
    )(f, W, A2, b2)


def _fin_body(aggp_ref, denp_ref, z_ref, w_ref, b_ref, zo_ref, so_ref):
    den = denp_ref[0] + denp_ref[1]
    agg = aggp_ref[0] + aggp_ref[1]
    pos = den > 0.0
    dsafe = jnp.where(pos, den, 1.0)
    zn = jnp.where(pos, jnp.maximum(agg / dsafe, 0.0), z_ref[...])
    zo_ref[...] = zn
    s = jnp.dot(zn, w_ref[...], preferred_element_type=jnp.float32)
    col = lax.broadcasted_iota(jnp.int32, s.shape, 1)
    bv = b_ref[...]
    so_ref[...] = s + jnp.where(col == 0, bv[0, 0], 0.0)


def _fin(aggp, denp, zprev, W2, b2):
    return pl.pallas_call(
        _fin_body,
        grid=(N // R,),
        in_specs=[
            pl.BlockSpec((2, R, D), lambda i: (0, i, 0)),
            pl.BlockSpec((2, R, 1), lambda i: (0, i, 0)),
            pl.BlockSpec((R, D), lambda i: (i, 0)),
            pl.BlockSpec((D, 2), lambda i: (0, 0)),
            pl.BlockSpec((1, 2), lambda i: (0, 0)),
        ],
        out_specs=[
            pl.BlockSpec((R, D), lambda i: (i, 0)),
            pl.BlockSpec((R, 2), lambda i: (i, 0)),
        ],
        out_shape=[
            jax.ShapeDtypeStruct((N, D), jnp.float32),
            jax.ShapeDtypeStruct((N, 2), jnp.float32),
        ],
    )(aggp, denp, zprev, W2, b2)


_SC_MESH = plsc.VectorSubcoreMesh(core_axis_name="c", subcore_axis_name="s")

_RPT = 624
_RLAST = _RPT * (NS - 1)
_DPT = 1000


def _chunks(total, step):
    off = 0
    while off < total:
        sz = min(step, total - off)
        yield off, sz
        off += sz


def _edge_body(ei_hbm, s12_hbm, z_hbm,
               agg_out, den_out,
               s12_v, src0, dst0, src1, dst1, src2, dst2, srcT, dstT,
               rows0, rows1, rows2, ex_v, srcS,
               agg_sh, den_sh,
               semI0, semI1, semI2, semG0, semG1, semG2):
    c = lax.axis_index("c")
    s = lax.axis_index("s")
    srcs = (src0, src1, src2)
    dsts = (dst0, dst1, dst2)
    rows = (rows0, rows1, rows2)
    semI = (semI0, semI1, semI2)
    semG = (semG0, semG1, semG2)

    zv = jnp.zeros((16,), jnp.float32)

    def zrow(i, carry):
        for j in range(D // 16):
            rows0[i, pl.ds(j * 16, 16)] = zv
        return carry

    lax.fori_loop(0, C, zrow, 0)

    def zex(g, carry):
        ex_v[pl.ds(g * 16, 16)] = zv
        return carry

    lax.fori_loop(0, C // 16, zex, 0)

    @pl.when(s < NS - 1)
    def _():
        for o, sz in _chunks(_RPT, C):
            pltpu.sync_copy(rows0.at[pl.ds(0, sz)],
                            agg_sh.at[pl.ds(s * _RPT + o, sz)])

    @pl.when(s == NS - 1)
    def _():
        for o, sz in _chunks(N - _RLAST, C):
            pltpu.sync_copy(rows0.at[pl.ds(0, sz)],
                            agg_sh.at[pl.ds(_RLAST + o, sz)])

    @pl.when(s < N // _DPT)
    def _():
        for o, sz in _chunks(_DPT, C):
            pltpu.sync_copy(ex_v.at[pl.ds(0, sz)],
                            den_sh.at[pl.ds(s * _DPT + o, sz)])

    pltpu.sync_copy(s12_hbm, s12_v)
    plsc.subcore_barrier()

    base = c * (E // NC) + s * EW

    def fetch_idx(j, si):
        off = base + j * C
        pltpu.async_copy(ei_hbm.at[pl.ds(off, C)], srcs[si], semI[si])
        pltpu.async_copy(ei_hbm.at[pl.ds(E + off, C)], dsts[si], semI[si])

    def drain_idx(si):
        pltpu.make_async_copy(ei_hbm.at[pl.ds(0, C)], srcs[si],
                              semI[si]).wait()
        pltpu.make_async_copy(ei_hbm.at[pl.ds(0, C)], dsts[si],
                              semI[si]).wait()

    def fire_gather(si):
        pltpu.async_copy(z_hbm.at[dsts[si]], rows[si], semG[si])

    def drain_gather(si):
        pltpu.make_async_copy(z_hbm.at[pl.ds(0, C)], rows[si],
                              semG[si]).wait()

    def process(k, r):
        @pl.when(k + 2 <= NCH - 1)
        def _():
            drain_idx((r + 2) % 3)
            fire_gather((r + 2) % 3)

        drain_gather(r)
        srcb = srcs[r]
        dstb = dsts[r]
        rowsb = rows[r]
        for g in range(C // 16):
            sl = pl.ds(g * 16, 16)
            srcv = srcb[sl]
            dstv = dstb[sl]
            srcS[sl] = srcv
            s1 = plsc.load_gather(s12_v, [srcv * 2])
            s2 = plsc.load_gather(s12_v, [dstv * 2 + 1])
            e = s1 + s2
            e = jnp.maximum(e, e * 0.01)
            ex_v[sl] = jnp.exp(e)
        pltpu.sync_copy(ex_v, den_sh.at[srcb], add=True)

        @pl.when(k + 3 <= NCH - 1)
        def _():
            fetch_idx(k + 3, r)

        def scaleg(g, carry):
            exv = ex_v[pl.ds(g * 16, 16)]
            r0 = g * 16
            for l in range(16):
                exi = exv[l]
                for j in range(D // 16):
                    slj = pl.ds(j * 16, 16)
                    rowsb[r0 + l, slj] = rowsb[r0 + l, slj] * exi
            return carry

        lax.fori_loop(0, C // 16, scaleg, 0)
        pltpu.sync_copy(rowsb, agg_sh.at[srcS], add=True)

    for j in range(3):
        fetch_idx(j, j)
    drain_idx(0)
    fire_gather(0)
    drain_idx(1)
    fire_gather(1)

    def triple(g, carry):
        for r in range(3):
            process(3 * g + r, r)
        return carry

    lax.fori_loop(0, NCH // 3, triple, 0)

    toff = base + NCH * C
    pltpu.sync_copy(ei_hbm.at[pl.ds(toff, TAIL)], srcT)
    pltpu.sync_copy(ei_hbm.at[pl.ds(E + toff, TAIL)], dstT)
    pltpu.async_copy(z_hbm.at[dstT], rows0.at[pl.ds(0, TAIL)], semG0).wait()
    srcv = srcT[pl.ds(0, TAIL)]
    dstv = dstT[pl.ds(0, TAIL)]
    s1 = plsc.load_gather(s12_v, [srcv * 2])
    s2 = plsc.load_gather(s12_v, [dstv * 2 + 1])
    e = s1 + s2
    e = jnp.maximum(e, e * 0.01)
    ext = jnp.exp(e)
    ex_v[pl.ds(0, TAIL)] = ext
    pltpu.sync_copy(ex_v.at[pl.ds(0, TAIL)], den_sh.at[srcT], add=True)
    for l in range(TAIL):
        exi = ext[l]
        for j in range(D // 16):
            slj = pl.ds(j * 16, 16)
            rows0[l, slj] = rows0[l, slj] * exi
    pltpu.sync_copy(rows0.at[pl.ds(0, TAIL)], agg_sh.at[srcT], add=True)
    plsc.subcore_barrier()

    @pl.when(s < NS - 1)
    def _():
        for o, sz in _chunks(_RPT, C):
            pltpu.sync_copy(agg_sh.at[pl.ds(s * _RPT + o, sz)],
                            rows0.at[pl.ds(0, sz)])
            pltpu.sync_copy(rows0.at[pl.ds(0, sz)],
                            agg_out.at[c, pl.ds(s * _RPT + o, sz)])

    @pl.when(s == NS - 1)
    def _():
        for o, sz in _chunks(N - _RLAST, C):
            pltpu.sync_copy(agg_sh.at[pl.ds(_RLAST + o, sz)],
                            rows0.at[pl.ds(0, sz)])
            pltpu.sync_copy(rows0.at[pl.ds(0, sz)],
                            agg_out.at[c, pl.ds(_RLAST + o, sz)])

    @pl.when(s < N // _DPT)
    def _():
        for o, sz in _chunks(_DPT, C):
            pltpu.sync_copy(den_sh.at[pl.ds(s * _DPT + o, sz)],
                            ex_v.at[pl.ds(0, sz)])
            pltpu.sync_copy(ex_v.at[pl.ds(0, sz)],
                            den_out.at[pl.ds(c * N + s * _DPT + o, sz)])


@functools.partial(
    pl.kernel,
    out_type=(
        jax.ShapeDtypeStruct((NC, N, D), jnp.float32),
        jax.ShapeDtypeStruct((NC * N,), jnp.float32),
    ),
    mesh=_SC_MESH,
    compiler_params=pltpu.CompilerParams(needs_layout_passes=False),
    scratch_types=[
        pltpu.VMEM((2 * N,), jnp.float32),
        pltpu.VMEM((C,), jnp.int32),
        pltpu.VMEM((C,), jnp.int32),
        pltpu.VMEM((C,), jnp.int32),
        pltpu.VMEM((C,), jnp.int32),
        pltpu.VMEM((C,), jnp.int32),
        pltpu.VMEM((C,), jnp.int32),
        pltpu.VMEM((TAIL,), jnp.int32),
        pltpu.VMEM((TAIL,), jnp.int32),
        pltpu.VMEM((C, D), jnp.float32),
        pltpu.VMEM((C, D), jnp.float32),
        pltpu.VMEM((C, D), jnp.float32),
        pltpu.VMEM((C,), jnp.float32),
        pltpu.VMEM((C,), jnp.int32),
        pltpu.VMEM_SHARED((N, D), jnp.float32),
        pltpu.VMEM_SHARED((N,), jnp.float32),
        pltpu.SemaphoreType.DMA,
        pltpu.SemaphoreType.DMA,
        pltpu.SemaphoreType.DMA,
        pltpu.SemaphoreType.DMA,
        pltpu.SemaphoreType.DMA,
        pltpu.SemaphoreType.DMA,
    ],
)
def _edge_pass(*refs):
    _edge_body(*refs)


_HPW = 1024 // NW


def _readout_body(t_hbm, hc_hbm, sal_hbm, prm_hbm, y_hbm,
                  t_v, hc_v, sal_v, prm_v, y_v):
    c = lax.axis_index("c")
    s = lax.axis_index("s")
    w = s * NC + c
    pltpu.sync_copy(t_hbm, t_v)
    pltpu.sync_copy(hc_hbm.at[pl.ds(w * _HPW, _HPW)], hc_v)
    pltpu.sync_copy(sal_hbm.at[pl.ds(w * _HPW, _HPW)], sal_v)
    pltpu.sync_copy(prm_hbm, prm_v)
    pv = prm_v[...]
    ws0 = pv[0]
    ws1 = pv[1]
    b0 = pv[2]
    b1 = pv[3]
    for g in range(_HPW // 16):
        sl = pl.ds(g * 16, 16)
        hc = hc_v[sl]
        sal = sal_v[sl]
        t0 = plsc.load_gather(t_v, [hc * 2])
        t1 = plsc.load_gather(t_v, [hc * 2 + 1])
        r0 = jnp.maximum(t0 + sal * ws0 + b0, 0.0)
        r1 = jnp.maximum(t1 + sal * ws1 + b1, 0.0)
        e0 = jnp.exp(r0)
        e1 = jnp.exp(r1)
        inv = 1.0 / (e0 + e1)
        idx = lax.iota(jnp.int32, 16) * 2 + g * 32
        plsc.store_scatter(y_v, [idx], e0 * inv)
        plsc.store_scatter(y_v, [idx + 1], e1 * inv)
    pltpu.sync_copy(y_v, y_hbm.at[pl.ds(w * 2 * _HPW, 2 * _HPW)])


@functools.partial(
    pl.kernel,
    out_type=jax.ShapeDtypeStruct((2048,), jnp.float32),
    mesh=_SC_MESH,
    compiler_params=pltpu.CompilerParams(needs_layout_passes=False),
    scratch_types=[
        pltpu.VMEM((2 * N,), jnp.float32),
        pltpu.VMEM((_HPW,), jnp.int32),
        pltpu.VMEM((_HPW,), jnp.float32),
        pltpu.VMEM((16,), jnp.float32),
        pltpu.VMEM((2 * _HPW,), jnp.float32),
    ],
)
def _readout(*refs):
    _readout_body(*refs)


def kernel(f, edge_index_cp, edge_index_hc, hc_ids, salary, W_fc,
           attn_W, attn_b, out_W, out_b):
    A2 = attn_W.reshape(2, D).T
    b2 = jnp.concatenate([attn_b, jnp.zeros((1,), jnp.float32)]).reshape(1, 2)
    zb2 = jnp.zeros((1, 2), jnp.float32)

    z0, s12 = _mm1(f, W_fc, A2, b2)
    aggp, denp = _edge_pass(edge_index_cp.reshape(-1), s12.reshape(-1), z0)
    z1, s12b = _fin(aggp, denp.reshape(NC, N, 1), z0, A2, b2)
    aggp2, denp2 = _edge_pass(edge_index_hc.reshape(-1), s12b.reshape(-1), z1)
    z2, t = _fin(aggp2, denp2.reshape(NC, N, 1), z1, out_W[:D], zb2)

    prm = jnp.concatenate([out_W[D], out_b, jnp.zeros((12,), jnp.float32)])
    yflat = _readout(t.reshape(-1), hc_ids, salary.reshape(-1), prm)
    return z2, yflat.reshape(-1, 2)


# speedup vs baseline: 35.9767x; 1.0271x over previous
import functools
import jax
import jax.numpy as jnp
from jax import lax
from jax.experimental import pallas as pl
from jax.experimental.pallas import tpu as pltpu
from jax.experimental.pallas import tpu_sc as plsc

N = 10000
D = 128
E = 320000
NC = 2
NS = 16
NW = NC * NS
EW = E // NW
C = 64
NCH = EW // C
TAIL = EW - NCH * C
R = 1000


def _mm_body(x_ref, w_ref, a_ref, b_ref, z_ref, s_ref):
    z = jnp.maximum(jnp.dot(x_ref[...], w_ref[...],
                            preferred_element_type=jnp.float32), 0.0)
    z_ref[...] = z
    s = jnp.dot(z, a_ref[...], preferred_element_type=jnp.float32)
    col = lax.broadcasted_iota(jnp.int32, s.shape, 1)
    bv = b_ref[...]
    s_ref[...] = s + jnp.where(col == 0, bv[0, 0], 0.0)


def _mm1(f, W, A2, b2):
    return pl.pallas_call(
        _mm_body,
        grid=(N // R,),
        in_specs=[
            pl.BlockSpec((R, D), lambda i: (i, 0)),
            pl.BlockSpec((D, D), lambda i: (0, 0)),
            pl.BlockSpec((D, 2), lambda i: (0, 0)),
            pl.BlockSpec((1, 2), lambda i: (0, 0)),
        ],
        out_specs=[
            pl.BlockSpec((R, D), lambda i: (i, 0)),
            pl.BlockSpec((R, 2), lambda i: (i, 0)),
        ],
        out_shape=[
            jax.ShapeDtypeStruct((N, D), jnp.float32),
            jax.ShapeDtypeStruct((N, 2), jnp.float32),
        ],
    )(f, W, A2, b2)


def _fin_body(aggp_ref, denp_ref, z_ref, w_ref, b_ref, zo_ref, so_ref):
    den = denp_ref[0] + denp_ref[1]
    agg = aggp_ref[0] + aggp_ref[1]
    pos = den > 0.0
    dsafe = jnp.where(pos, den, 1.0)
    zn = jnp.where(pos, jnp.maximum(agg / dsafe, 0.0), z_ref[...])
    zo_ref[...] = zn
    s = jnp.dot(zn, w_ref[...], preferred_element_type=jnp.float32)
    col = lax.broadcasted_iota(jnp.int32, s.shape, 1)
    bv = b_ref[...]
    so_ref[...] = s + jnp.where(col == 0, bv[0, 0], 0.0)


def _fin(aggp, denp, zprev, W2, b2):
    return pl.pallas_call(
        _fin_body,
        grid=(N // R,),
        in_specs=[
            pl.BlockSpec((2, R, D), lambda i: (0, i, 0)),
            pl.BlockSpec((2, R, 1), lambda i: (0, i, 0)),
            pl.BlockSpec((R, D), lambda i: (i, 0)),
            pl.BlockSpec((D, 2), lambda i: (0, 0)),
            pl.BlockSpec((1, 2), lambda i: (0, 0)),
        ],
        out_specs=[
            pl.BlockSpec((R, D), lambda i: (i, 0)),
            pl.BlockSpec((R, 2), lambda i: (i, 0)),
        ],
        out_shape=[
            jax.ShapeDtypeStruct((N, D), jnp.float32),
            jax.ShapeDtypeStruct((N, 2), jnp.float32),
        ],
    )(aggp, denp, zprev, W2, b2)


_SC_MESH = plsc.VectorSubcoreMesh(core_axis_name="c", subcore_axis_name="s")

_RPT = 624
_RLAST = _RPT * (NS - 1)
_DPT = 1000


def _chunks(total, step):
    off = 0
    while off < total:
        sz = min(step, total - off)
        yield off, sz
        off += sz


def _edge_body(ei_hbm, s12_hbm, z_hbm,
               agg_out, den_out,
               s12_v, src0, dst0, src1, dst1, src2, dst2, srcT, dstT,
               rows0, rows1, rows2, ex_v, srcS,
               agg_sh, den_sh,
               semI0, semI1, semI2, semG0, semG1, semG2):
    c = lax.axis_index("c")
    s = lax.axis_index("s")
    srcs = (src0, src1, src2)
    dsts = (dst0, dst1, dst2)
    rows = (rows0, rows1, rows2)
    semI = (semI0, semI1, semI2)
    semG = (semG0, semG1, semG2)

    zv = jnp.zeros((16,), jnp.float32)

    def zrow(i, carry):
        for j in range(D // 16):
            rows0[i, pl.ds(j * 16, 16)] = zv
        return carry

    lax.fori_loop(0, C, zrow, 0)

    def zex(g, carry):
        ex_v[pl.ds(g * 16, 16)] = zv
        return carry

    lax.fori_loop(0, C // 16, zex, 0)

    @pl.when(s < NS - 1)
    def _():
        for o, sz in _chunks(_RPT, C):
            pltpu.async_copy(rows0.at[pl.ds(0, sz)],
                             agg_sh.at[pl.ds(s * _RPT + o, sz)], semI0)

    @pl.when(s == NS - 1)
    def _():
        for o, sz in _chunks(N - _RLAST, C):
            pltpu.async_copy(rows0.at[pl.ds(0, sz)],
                             agg_sh.at[pl.ds(_RLAST + o, sz)], semI0)

    @pl.when(s < N // _DPT)
    def _():
        for o, sz in _chunks(_DPT, C):
            pltpu.sync_copy(ex_v.at[pl.ds(0, sz)],
                            den_sh.at[pl.ds(s * _DPT + o, sz)])

    pltpu.sync_copy(s12_hbm, s12_v)

    @pl.when(s < NS - 1)
    def _():
        for o, sz in _chunks(_RPT, C):
            pltpu.make_async_copy(rows0.at[pl.ds(0, sz)],
                                  agg_sh.at[pl.ds(o, sz)], semI0).wait()

    @pl.when(s == NS - 1)
    def _():
        for o, sz in _chunks(N - _RLAST, C):
            pltpu.make_async_copy(rows0.at[pl.ds(0, sz)],
                                  agg_sh.at[pl.ds(o, sz)], semI0).wait()

    plsc.subcore_barrier()

    base = c * (E // NC) + s * EW

    def fetch_idx(j, si):
        off = base + j * C
        pltpu.async_copy(ei_hbm.at[pl.ds(off, C)], srcs[si], semI[si])
        pltpu.async_copy(ei_hbm.at[pl.ds(E + off, C)], dsts[si], semI[si])

    def drain_idx(si):
        pltpu.make_async_copy(ei_hbm.at[pl.ds(0, C)], srcs[si],
                              semI[si]).wait()
        pltpu.make_async_copy(ei_hbm.at[pl.ds(0, C)], dsts[si],
                              semI[si]).wait()

    def fire_gather(si):
        pltpu.async_copy(z_hbm.at[dsts[si]], rows[si], semG[si])

    def drain_gather(si):
        pltpu.make_async_copy(z_hbm.at[pl.ds(0, C)], rows[si],
                              semG[si]).wait()

    def process(k, r):
        @pl.when(k + 2 <= NCH - 1)
        def _():
            drain_idx((r + 2) % 3)
            fire_gather((r + 2) % 3)

        drain_gather(r)
        srcb = srcs[r]
        dstb = dsts[r]
        rowsb = rows[r]
        for g in range(C // 16):
            sl = pl.ds(g * 16, 16)
            srcv = srcb[sl]
            dstv = dstb[sl]
            srcS[sl] = srcv
            s1 = plsc.load_gather(s12_v, [srcv * 2])
            s2 = plsc.load_gather(s12_v, [dstv * 2 + 1])
            e = s1 + s2
            e = jnp.maximum(e, e * 0.01)
            ex_v[sl] = jnp.exp(e)
        pltpu.sync_copy(ex_v, den_sh.at[srcb], add=True)

        @pl.when(k + 3 <= NCH - 1)
        def _():
            fetch_idx(k + 3, r)

        def scaleg(g, carry):
            exv = ex_v[pl.ds(g * 16, 16)]
            r0 = g * 16
            for l in range(16):
                exi = exv[l]
                for j in range(D // 16):
                    slj = pl.ds(j * 16, 16)
                    rowsb[r0 + l, slj] = rowsb[r0 + l, slj] * exi
            return carry

        lax.fori_loop(0, C // 16, scaleg, 0)
        pltpu.sync_copy(rowsb, agg_sh.at[srcS], add=True)

    for j in range(3):
        fetch_idx(j, j)
    drain_idx(0)
    fire_gather(0)
    drain_idx(1)
    fire_gather(1)

    def triple(g, carry):
        for r in range(3):
            process(3 * g + r, r)
        return carry

    lax.fori_loop(0, NCH // 3, triple, 0)

    toff = base + NCH * C
    pltpu.sync_copy(ei_hbm.at[pl.ds(toff, TAIL)], srcT)
    pltpu.sync_copy(ei_hbm.at[pl.ds(E + toff, TAIL)], dstT)
    pltpu.async_copy(z_hbm.at[dstT], rows0.at[pl.ds(0, TAIL)], semG0).wait()
    srcv = srcT[pl.ds(0, TAIL)]
    dstv = dstT[pl.ds(0, TAIL)]
    s1 = plsc.load_gather(s12_v, [srcv * 2])
    s2 = plsc.load_gather(s12_v, [dstv * 2 + 1])
    e = s1 + s2
    e = jnp.maximum(e, e * 0.01)
    ext = jnp.exp(e)
    ex_v[pl.ds(0, TAIL)] = ext
    pltpu.sync_copy(ex_v.at[pl.ds(0, TAIL)], den_sh.at[srcT], add=True)
    for l in range(TAIL):
        exi = ext[l]
        for j in range(D // 16):
            slj = pl.ds(j * 16, 16)
            rows0[l, slj] = rows0[l, slj] * exi
    pltpu.sync_copy(rows0.at[pl.ds(0, TAIL)], agg_sh.at[srcT], add=True)
    plsc.subcore_barrier()

    def drain_rows(r0v, nrows):
        cps = []
        chs = list(_chunks(nrows, C))
        for i, (o, sz) in enumerate(chs):
            buf = rows[i % 2]
            if i >= 2:
                cps[i - 2].wait()
            pltpu.sync_copy(agg_sh.at[pl.ds(r0v + o, sz)],
                            buf.at[pl.ds(0, sz)])
            cps.append(pltpu.async_copy(
                buf.at[pl.ds(0, sz)],
                agg_out.at[c, pl.ds(r0v + o, sz)], semG[i % 2]))
        for cp in cps[-2:]:
            cp.wait()

    @pl.when(s < NS - 1)
    def _():
        drain_rows(s * _RPT, _RPT)

    @pl.when(s == NS - 1)
    def _():
        drain_rows(_RLAST, N - _RLAST)

    @pl.when(s < N // _DPT)
    def _():
        for o, sz in _chunks(_DPT, C):
            pltpu.sync_copy(den_sh.at[pl.ds(s * _DPT + o, sz)],
                            ex_v.at[pl.ds(0, sz)])
            pltpu.sync_copy(ex_v.at[pl.ds(0, sz)],
                            den_out.at[pl.ds(c * N + s * _DPT + o, sz)])


@functools.partial(
    pl.kernel,
    out_type=(
        jax.ShapeDtypeStruct((NC, N, D), jnp.float32),
        jax.ShapeDtypeStruct((NC * N,), jnp.float32),
    ),
    mesh=_SC_MESH,
    compiler_params=pltpu.CompilerParams(needs_layout_passes=False),
    scratch_types=[
        pltpu.VMEM((2 * N,), jnp.float32),
        pltpu.VMEM((C,), jnp.int32),
        pltpu.VMEM((C,), jnp.int32),
        pltpu.VMEM((C,), jnp.int32),
        pltpu.VMEM((C,), jnp.int32),
        pltpu.VMEM((C,), jnp.int32),
        pltpu.VMEM((C,), jnp.int32),
        pltpu.VMEM((TAIL,), jnp.int32),
        pltpu.VMEM((TAIL,), jnp.int32),
        pltpu.VMEM((C, D), jnp.float32),
        pltpu.VMEM((C, D), jnp.float32),
        pltpu.VMEM((C, D), jnp.float32),
        pltpu.VMEM((C,), jnp.float32),
        pltpu.VMEM((C,), jnp.int32),
        pltpu.VMEM_SHARED((N, D), jnp.float32),
        pltpu.VMEM_SHARED((N,), jnp.float32),
        pltpu.SemaphoreType.DMA,
        pltpu.SemaphoreType.DMA,
        pltpu.SemaphoreType.DMA,
        pltpu.SemaphoreType.DMA,
        pltpu.SemaphoreType.DMA,
        pltpu.SemaphoreType.DMA,
    ],
)
def _edge_pass(*refs):
    _edge_body(*refs)


_HPW = 1024 // NW


def _readout_body(t_hbm, hc_hbm, sal_hbm, prm_hbm, y_hbm,
                  t_v, hc_v, sal_v, prm_v, y_v):
    c = lax.axis_index("c")
    s = lax.axis_index("s")
    w = s * NC + c
    pltpu.sync_copy(t_hbm, t_v)
    pltpu.sync_copy(hc_hbm.at[pl.ds(w * _HPW, _HPW)], hc_v)
    pltpu.sync_copy(sal_hbm.at[pl.ds(w * _HPW, _HPW)], sal_v)
    pltpu.sync_copy(prm_hbm, prm_v)
    pv = prm_v[...]
    ws0 = pv[0]
    ws1 = pv[1]
    b0 = pv[2]
    b1 = pv[3]
    for g in range(_HPW // 16):
        sl = pl.ds(g * 16, 16)
        hc = hc_v[sl]
        sal = sal_v[sl]
        t0 = plsc.load_gather(t_v, [hc * 2])
        t1 = plsc.load_gather(t_v, [hc * 2 + 1])
        r0 = jnp.maximum(t0 + sal * ws0 + b0, 0.0)
        r1 = jnp.maximum(t1 + sal * ws1 + b1, 0.0)
        e0 = jnp.exp(r0)
        e1 = jnp.exp(r1)
        inv = 1.0 / (e0 + e1)
        idx = lax.iota(jnp.int32, 16) * 2 + g * 32
        plsc.store_scatter(y_v, [idx], e0 * inv)
        plsc.store_scatter(y_v, [idx + 1], e1 * inv)
    pltpu.sync_copy(y_v, y_hbm.at[pl.ds(w * 2 * _HPW, 2 * _HPW)])


@functools.partial(
    pl.kernel,
    out_type=jax.ShapeDtypeStruct((2048,), jnp.float32),
    mesh=_SC_MESH,
    compiler_params=pltpu.CompilerParams(needs_layout_passes=False),
    scratch_types=[
        pltpu.VMEM((2 * N,), jnp.float32),
        pltpu.VMEM((_HPW,), jnp.int32),
        pltpu.VMEM((_HPW,), jnp.float32),
        pltpu.VMEM((16,), jnp.float32),
        pltpu.VMEM((2 * _HPW,), jnp.float32),
    ],
)
def _readout(*refs):
    _readout_body(*refs)


def kernel(f, edge_index_cp, edge_index_hc, hc_ids, salary, W_fc,
           attn_W, attn_b, out_W, out_b):
    A2 = attn_W.reshape(2, D).T
    b2 = jnp.concatenate([attn_b, jnp.zeros((1,), jnp.float32)]).reshape(1, 2)
    zb2 = jnp.zeros((1, 2), jnp.float32)

    z0, s12 = _mm1(f, W_fc, A2, b2)
    aggp, denp = _edge_pass(edge_index_cp.reshape(-1), s12.reshape(-1), z0)
    z1, s12b = _fin(aggp, denp.reshape(NC, N, 1), z0, A2, b2)
    aggp2, denp2 = _edge_pass(edge_index_hc.reshape(-1), s12b.reshape(-1), z1)
    z2, t = _fin(aggp2, denp2.reshape(NC, N, 1), z1, out_W[:D], zb2)

    prm = jnp.concatenate([out_W[D], out_b, jnp.zeros((12,), jnp.float32)])
    yflat = _readout(t.reshape(-1), hc_ids, salary.reshape(-1), prm)
    return z2, yflat.reshape(-1, 2)
